# Initial kernel scaffold; baseline (speedup 1.0000x reference)
import jax
import jax.numpy as jnp
from jax.experimental import pallas as pl


def kernel(z, edge, W1, b1, W2, b2):
    raise NotImplementedError("write your pallas kernel here")


# speedup vs baseline: 1.8004x; 1.8004x over previous
import functools

import jax
import jax.numpy as jnp
from jax import lax
from jax.experimental import pallas as pl
from jax.experimental.pallas import tpu as pltpu
from jax.experimental.pallas import tpu_sc as plsc

N_NODES = 10000
N_EDGES = 160000
D = 256
HID = 512

NC = 2
NS = 16
NW = NC * NS
E_PER_W = N_EDGES // NW
CHUNK = 200
N_CHUNKS = E_PER_W // CHUNK
LANES = 16


def _make_gather_mul():
    mesh = plsc.VectorSubcoreMesh(core_axis_name="c", subcore_axis_name="s")

    @functools.partial(
        pl.kernel,
        mesh=mesh,
        out_type=jax.ShapeDtypeStruct((N_EDGES, D), jnp.float32),
        scratch_types=[
            pltpu.VMEM((CHUNK,), jnp.int32),
            pltpu.VMEM((CHUNK,), jnp.int32),
            pltpu.VMEM((CHUNK, D), jnp.float32),
            pltpu.VMEM((CHUNK, D), jnp.float32),
            pltpu.SemaphoreType.DMA,
            pltpu.SemaphoreType.DMA,
        ],
    )
    def gather_mul(z_hbm, idx0_hbm, idx1_hbm, out_hbm,
                   idx0_v, idx1_v, a_v, b_v, sem0, sem1):
        wid = lax.axis_index("s") * NC + lax.axis_index("c")
        base = wid * E_PER_W

        def chunk_body(ci, carry):
            off = base + ci * CHUNK
            pltpu.sync_copy(idx0_hbm.at[pl.ds(off, CHUNK)], idx0_v)
            pltpu.sync_copy(idx1_hbm.at[pl.ds(off, CHUNK)], idx1_v)
            cp0 = pltpu.async_copy(z_hbm.at[idx0_v], a_v, sem0)
            cp1 = pltpu.async_copy(z_hbm.at[idx1_v], b_v, sem1)
            cp0.wait()
            cp1.wait()

            def row_body(r, c2):
                for j in range(D // LANES):
                    sl = pl.ds(j * LANES, LANES)
                    a_v[r, sl] = a_v[r, sl] * b_v[r, sl]
                return c2

            lax.fori_loop(0, CHUNK, row_body, 0)
            pltpu.sync_copy(a_v, out_hbm.at[pl.ds(off, CHUNK)])
            return carry

        lax.fori_loop(0, N_CHUNKS, chunk_body, 0)

    return gather_mul


_gather_mul_cache = []


def _gather_mul(z, idx0, idx1):
    if not _gather_mul_cache:
        _gather_mul_cache.append(_make_gather_mul())
    return _gather_mul_cache[0](z, idx0, idx1)


BE = 1600


def _mlp_body(x_ref, w1_ref, b1_ref, w2t_ref, b2_ref, o_ref):
    x = x_ref[...]
    h = jnp.dot(x, w1_ref[...], preferred_element_type=jnp.float32)
    h = jnp.maximum(h + b1_ref[...], 0.0)
    logits = jnp.sum(h * w2t_ref[...], axis=1, keepdims=True) + b2_ref[...]
    o_ref[...] = jax.nn.sigmoid(logits)


def _mlp(x, W1, b1, W2, b2):
    grid = N_EDGES // BE
    return pl.pallas_call(
        _mlp_body,
        grid=(grid,),
        in_specs=[
            pl.BlockSpec((BE, D), lambda i: (i, 0)),
            pl.BlockSpec((D, HID), lambda i: (0, 0)),
            pl.BlockSpec((1, HID), lambda i: (0, 0)),
            pl.BlockSpec((1, HID), lambda i: (0, 0)),
            pl.BlockSpec((1, 1), lambda i: (0, 0)),
        ],
        out_specs=pl.BlockSpec((BE, 1), lambda i: (i, 0)),
        out_shape=jax.ShapeDtypeStruct((N_EDGES, 1), jnp.float32),
        compiler_params=pltpu.CompilerParams(
            dimension_semantics=("arbitrary",),
        ),
    )(x, W1, b1, W2, b2)


def kernel(z, edge, W1, b1, W2, b2):
    edge = edge.astype(jnp.int32)
    x = _gather_mul(z, edge[0], edge[1])
    return _mlp(x, W1, b1.reshape(1, HID), W2.reshape(1, HID),
                b2.reshape(1, 1))


# speedup vs baseline: 2.1633x; 1.2016x over previous
import functools

import jax
import jax.numpy as jnp
import numpy as np
from jax import lax
from jax.experimental import pallas as pl
from jax.experimental.pallas import tpu as pltpu
from jax.experimental.pallas import tpu_sc as plsc

N_NODES = 10000
N_EDGES = 160000
D = 256
HID = 512

NC = 2
NS = 16
NW = NC * NS
E_PER_W = N_EDGES // NW
CHUNK = 40
N_CHUNKS = E_PER_W // CHUNK
LANES = 16

def _make_gather_mul():
    mesh = plsc.VectorSubcoreMesh(core_axis_name="c", subcore_axis_name="s")

    @functools.partial(
        pl.kernel,
        mesh=mesh,
        out_type=jax.ShapeDtypeStruct((N_EDGES, D), jnp.float32),
        scratch_types=[
            pltpu.VMEM((N_CHUNKS, CHUNK), jnp.int32),
            pltpu.VMEM((N_CHUNKS, CHUNK), jnp.int32),
            pltpu.VMEM((2, CHUNK, D), jnp.float32),
            pltpu.VMEM((2, CHUNK, D), jnp.float32),
            pltpu.VMEM((2, CHUNK, D), jnp.float32),
            pltpu.SemaphoreType.DMA,
            pltpu.SemaphoreType.DMA,
            pltpu.SemaphoreType.DMA,
            pltpu.SemaphoreType.DMA,
            pltpu.SemaphoreType.DMA,
            pltpu.SemaphoreType.DMA,
        ],
    )
    def gather_mul(z_hbm, idx0_hbm, idx1_hbm, out_hbm,
                   idx0_v, idx1_v, a_v, b_v, o_v,
                   sa0, sa1, sb0, sb1, so0, so1):
        wid = lax.axis_index("s") * NC + lax.axis_index("c")
        base = wid * E_PER_W
        sa = (sa0, sa1)
        sb = (sb0, sb1)
        so = (so0, so1)

        pltpu.sync_copy(idx0_hbm.at[wid], idx0_v)
        pltpu.sync_copy(idx1_hbm.at[wid], idx1_v)

        def start_gather(ci, p):
            pltpu.async_copy(z_hbm.at[idx0_v.at[ci]], a_v.at[p], sa[p])
            pltpu.async_copy(z_hbm.at[idx1_v.at[ci]], b_v.at[p], sb[p])

        def wait_gather(ci, p):
            pltpu.make_async_copy(z_hbm.at[idx0_v.at[ci]], a_v.at[p], sa[p]).wait()
            pltpu.make_async_copy(z_hbm.at[idx1_v.at[ci]], b_v.at[p], sb[p]).wait()

        def out_slice(ci):
            return out_hbm.at[pl.ds(base + ci * CHUNK, CHUNK)]

        def mul_pack(p):
            def row_body(r, c):
                for k in range(D // LANES):
                    sl = pl.ds(LANES * k, LANES)
                    o_v[p, r, sl] = a_v[p, r, sl] * b_v[p, r, sl]
                return c
            lax.fori_loop(0, CHUNK, row_body, 0)

        def wait_wb(ci, p):
            pltpu.make_async_copy(o_v.at[p], out_slice(ci), so[p]).wait()

        def process(ci, p, first):
            if not first:
                wait_wb(ci - 2, p)
            mul_pack(p)
            pltpu.async_copy(o_v.at[p], out_slice(ci), so[p])

        start_gather(0, 0)

        def pair_body(i, carry):
            g0 = 2 * i
            wait_gather(g0, 0)
            start_gather(g0 + 1, 1)

            @pl.when(i > 0)
            def _():
                wait_wb(g0 - 2, 0)
            mul_pack(0)
            pltpu.async_copy(o_v.at[0], out_slice(g0), so[0])

            wait_gather(g0 + 1, 1)
            start_gather(g0 + 2, 0)

            @pl.when(i > 0)
            def _():
                wait_wb(g0 - 1, 1)
            mul_pack(1)
            pltpu.async_copy(o_v.at[1], out_slice(g0 + 1), so[1])
            return carry

        lax.fori_loop(0, (N_CHUNKS - 1) // 2, pair_body, 0)

        last = N_CHUNKS - 1
        wait_gather(last, 0)
        wait_wb(last - 2, 0)
        mul_pack(0)
        pltpu.async_copy(o_v.at[0], out_slice(last), so[0])
        wait_wb(last, 0)
        wait_wb(last - 1, 1)

    return gather_mul


_gather_mul_cache = []


def _gather_mul(z, idx0, idx1):
    if not _gather_mul_cache:
        _gather_mul_cache.append(_make_gather_mul())
    return _gather_mul_cache[0](z, idx0, idx1)


BE = 1600


def _mlp_body(x_ref, w1_ref, b1_ref, w2t_ref, b2_ref, o_ref):
    x = x_ref[...].astype(jnp.bfloat16)
    h = jnp.dot(x, w1_ref[...], preferred_element_type=jnp.float32)
    h = jnp.maximum(h + b1_ref[...], 0.0)
    logits = jnp.sum(h * w2t_ref[...], axis=1, keepdims=True) + b2_ref[...]
    o_ref[...] = jax.nn.sigmoid(logits)


def _mlp(x, W1, b1, W2, b2):
    grid = N_EDGES // BE
    return pl.pallas_call(
        _mlp_body,
        grid=(grid,),
        in_specs=[
            pl.BlockSpec((BE, D), lambda i: (i, 0)),
            pl.BlockSpec((D, HID), lambda i: (0, 0)),
            pl.BlockSpec((1, HID), lambda i: (0, 0)),
            pl.BlockSpec((1, HID), lambda i: (0, 0)),
            pl.BlockSpec((1, 1), lambda i: (0, 0)),
        ],
        out_specs=pl.BlockSpec((BE, 1), lambda i: (i, 0)),
        out_shape=jax.ShapeDtypeStruct((N_EDGES, 1), jnp.float32),
        compiler_params=pltpu.CompilerParams(
            dimension_semantics=("arbitrary",),
        ),
    )(x, W1, b1, W2, b2)


def kernel(z, edge, W1, b1, W2, b2):
    edge = edge.astype(jnp.int32)
    idx0 = edge[0].reshape(NW, N_CHUNKS, CHUNK)
    idx1 = edge[1].reshape(NW, N_CHUNKS, CHUNK)
    x = _gather_mul(z, idx0, idx1)
    return _mlp(x, W1.astype(jnp.bfloat16), b1.reshape(1, HID),
                W2.reshape(1, HID), b2.reshape(1, 1))


# speedup vs baseline: 2.4064x; 1.1124x over previous
import functools

import jax
import jax.numpy as jnp
import numpy as np
from jax import lax
from jax.experimental import pallas as pl
from jax.experimental.pallas import tpu as pltpu
from jax.experimental.pallas import tpu_sc as plsc

N_NODES = 10000
N_EDGES = 160000
D = 256
HID = 512

NC = 2
NS = 16
NW = NC * NS
E_PER_W = N_EDGES // NW
CHUNK = 40
N_CHUNKS = E_PER_W // CHUNK
LANES = 16

def _make_gather_mul():
    mesh = plsc.VectorSubcoreMesh(core_axis_name="c", subcore_axis_name="s")

    @functools.partial(
        pl.kernel,
        mesh=mesh,
        out_type=jax.ShapeDtypeStruct((N_EDGES, D), jnp.float32),
        scratch_types=[
            pltpu.VMEM((N_CHUNKS, CHUNK), jnp.int32),
            pltpu.VMEM((N_CHUNKS, CHUNK), jnp.int32),
            pltpu.VMEM((2, CHUNK, D), jnp.float32),
            pltpu.VMEM((2, CHUNK, D), jnp.float32),
            pltpu.VMEM((2, CHUNK, D), jnp.float32),
            pltpu.SemaphoreType.DMA,
            pltpu.SemaphoreType.DMA,
            pltpu.SemaphoreType.DMA,
            pltpu.SemaphoreType.DMA,
            pltpu.SemaphoreType.DMA,
            pltpu.SemaphoreType.DMA,
        ],
    )
    def gather_mul(z_hbm, idx0_hbm, idx1_hbm, out_hbm,
                   idx0_v, idx1_v, a_v, b_v, o_v,
                   sa0, sa1, sb0, sb1, so0, so1):
        wid = lax.axis_index("s") * NC + lax.axis_index("c")
        base = wid * E_PER_W
        sa = (sa0, sa1)
        sb = (sb0, sb1)
        so = (so0, so1)

        pltpu.sync_copy(idx0_hbm.at[wid], idx0_v)
        pltpu.sync_copy(idx1_hbm.at[wid], idx1_v)

        def start_gather(ci, p):
            pltpu.async_copy(z_hbm.at[idx0_v.at[ci]], a_v.at[p], sa[p])
            pltpu.async_copy(z_hbm.at[idx1_v.at[ci]], b_v.at[p], sb[p])

        def wait_gather(ci, p):
            pltpu.make_async_copy(z_hbm.at[idx0_v.at[ci]], a_v.at[p], sa[p]).wait()
            pltpu.make_async_copy(z_hbm.at[idx1_v.at[ci]], b_v.at[p], sb[p]).wait()

        def out_slice(ci):
            return out_hbm.at[pl.ds(base + ci * CHUNK, CHUNK)]

        def mul_pack(p):
            def row_body(r, c):
                for k in range(D // LANES):
                    sl = pl.ds(LANES * k, LANES)
                    o_v[p, r, sl] = a_v[p, r, sl] * b_v[p, r, sl]
                return c
            lax.fori_loop(0, CHUNK, row_body, 0)

        def wait_wb(ci, p):
            pltpu.make_async_copy(o_v.at[p], out_slice(ci), so[p]).wait()

        def process(ci, p, first):
            if not first:
                wait_wb(ci - 2, p)
            mul_pack(p)
            pltpu.async_copy(o_v.at[p], out_slice(ci), so[p])

        start_gather(0, 0)

        def pair_body(i, carry):
            g0 = 2 * i
            wait_gather(g0, 0)
            start_gather(g0 + 1, 1)

            @pl.when(i > 0)
            def _():
                wait_wb(g0 - 2, 0)
            mul_pack(0)
            pltpu.async_copy(o_v.at[0], out_slice(g0), so[0])

            wait_gather(g0 + 1, 1)
            start_gather(g0 + 2, 0)

            @pl.when(i > 0)
            def _():
                wait_wb(g0 - 1, 1)
            mul_pack(1)
            pltpu.async_copy(o_v.at[1], out_slice(g0 + 1), so[1])
            return carry

        lax.fori_loop(0, (N_CHUNKS - 1) // 2, pair_body, 0)

        last = N_CHUNKS - 1
        wait_gather(last, 0)
        wait_wb(last - 2, 0)
        mul_pack(0)
        pltpu.async_copy(o_v.at[0], out_slice(last), so[0])
        wait_wb(last, 0)
        wait_wb(last - 1, 1)

    return gather_mul


_gather_mul_cache = []


def _gather_mul(z, idx0, idx1):
    if not _gather_mul_cache:
        _gather_mul_cache.append(_make_gather_mul())
    return _gather_mul_cache[0](z, idx0, idx1)


BE = 4000


def _mlp_body(x_ref, w1_ref, b1_ref, w2t_ref, b2_ref, o_ref):
    x = x_ref[...].astype(jnp.bfloat16)
    h = jnp.dot(x, w1_ref[...], preferred_element_type=jnp.float32)
    h = jnp.maximum(h + b1_ref[...], 0.0)
    logits = jnp.sum(h * w2t_ref[...], axis=1, keepdims=True) + b2_ref[...]
    o_ref[...] = jax.nn.sigmoid(logits)


def _mlp(x, W1, b1, W2, b2):
    grid = N_EDGES // BE
    return pl.pallas_call(
        _mlp_body,
        grid=(grid,),
        in_specs=[
            pl.BlockSpec((BE, D), lambda i: (i, 0)),
            pl.BlockSpec((D, HID), lambda i: (0, 0)),
            pl.BlockSpec((1, HID), lambda i: (0, 0)),
            pl.BlockSpec((1, HID), lambda i: (0, 0)),
            pl.BlockSpec((1, 1), lambda i: (0, 0)),
        ],
        out_specs=pl.BlockSpec((BE, 1), lambda i: (i, 0)),
        out_shape=jax.ShapeDtypeStruct((N_EDGES, 1), jnp.float32),
        compiler_params=pltpu.CompilerParams(
            dimension_semantics=("arbitrary",),
        ),
    )(x, W1, b1, W2, b2)


def kernel(z, edge, W1, b1, W2, b2):
    edge = edge.astype(jnp.int32)
    idx0 = edge[0].reshape(NW, N_CHUNKS, CHUNK)
    idx1 = edge[1].reshape(NW, N_CHUNKS, CHUNK)
    x = _gather_mul(z, idx0, idx1)
    return _mlp(x, W1.astype(jnp.bfloat16), b1.reshape(1, HID),
                W2.reshape(1, HID), b2.reshape(1, 1))


# speedup vs baseline: 2.4944x; 1.0365x over previous
import functools

import jax
import jax.numpy as jnp
import numpy as np
from jax import lax
from jax.experimental import pallas as pl
from jax.experimental.pallas import tpu as pltpu
from jax.experimental.pallas import tpu_sc as plsc

N_NODES = 10000
N_EDGES = 160000
D = 256
HID = 512

NC = 2
NS = 16
NW = NC * NS
E_PER_W = N_EDGES // NW
CHUNK = 40
N_CHUNKS = E_PER_W // CHUNK
LANES = 16

def _make_gather_mul():
    mesh = plsc.VectorSubcoreMesh(core_axis_name="c", subcore_axis_name="s")

    @functools.partial(
        pl.kernel,
        mesh=mesh,
        out_type=jax.ShapeDtypeStruct((N_EDGES, D), jnp.float32),
        scratch_types=[
            pltpu.VMEM((N_CHUNKS, CHUNK), jnp.int32),
            pltpu.VMEM((N_CHUNKS, CHUNK), jnp.int32),
            pltpu.VMEM((2, CHUNK, D), jnp.float32),
            pltpu.VMEM((2, CHUNK, D), jnp.float32),
            pltpu.VMEM((2, CHUNK, D), jnp.float32),
            pltpu.SemaphoreType.DMA,
            pltpu.SemaphoreType.DMA,
            pltpu.SemaphoreType.DMA,
            pltpu.SemaphoreType.DMA,
            pltpu.SemaphoreType.DMA,
            pltpu.SemaphoreType.DMA,
        ],
    )
    def gather_mul(z_hbm, idx0_hbm, idx1_hbm, out_hbm,
                   idx0_v, idx1_v, a_v, b_v, o_v,
                   sa0, sa1, sb0, sb1, so0, so1):
        wid = lax.axis_index("s") * NC + lax.axis_index("c")
        base = wid * E_PER_W
        sa = (sa0, sa1)
        sb = (sb0, sb1)
        so = (so0, so1)

        pltpu.sync_copy(idx0_hbm.at[wid], idx0_v)
        pltpu.sync_copy(idx1_hbm.at[wid], idx1_v)

        def start_gather(ci, p):
            pltpu.async_copy(z_hbm.at[idx0_v.at[ci]], a_v.at[p], sa[p])
            pltpu.async_copy(z_hbm.at[idx1_v.at[ci]], b_v.at[p], sb[p])

        def wait_gather(ci, p):
            pltpu.make_async_copy(z_hbm.at[idx0_v.at[ci]], a_v.at[p], sa[p]).wait()
            pltpu.make_async_copy(z_hbm.at[idx1_v.at[ci]], b_v.at[p], sb[p]).wait()

        def out_slice(ci):
            return out_hbm.at[pl.ds(base + ci * CHUNK, CHUNK)]

        def mul_pack(p):
            def row_body(r, c):
                for k in range(D // LANES):
                    sl = pl.ds(LANES * k, LANES)
                    o_v[p, r, sl] = a_v[p, r, sl] * b_v[p, r, sl]
                return c
            lax.fori_loop(0, CHUNK, row_body, 0)

        def wait_wb(ci, p):
            pltpu.make_async_copy(o_v.at[p], out_slice(ci), so[p]).wait()

        def process(ci, p, first):
            if not first:
                wait_wb(ci - 2, p)
            mul_pack(p)
            pltpu.async_copy(o_v.at[p], out_slice(ci), so[p])

        start_gather(0, 0)

        def pair_body(i, carry):
            g0 = 2 * i
            wait_gather(g0, 0)
            start_gather(g0 + 1, 1)

            @pl.when(i > 0)
            def _():
                wait_wb(g0 - 2, 0)
            mul_pack(0)
            pltpu.async_copy(o_v.at[0], out_slice(g0), so[0])

            wait_gather(g0 + 1, 1)
            start_gather(g0 + 2, 0)

            @pl.when(i > 0)
            def _():
                wait_wb(g0 - 1, 1)
            mul_pack(1)
            pltpu.async_copy(o_v.at[1], out_slice(g0 + 1), so[1])
            return carry

        lax.fori_loop(0, (N_CHUNKS - 1) // 2, pair_body, 0)

        last = N_CHUNKS - 1
        wait_gather(last, 0)
        wait_wb(last - 2, 0)
        mul_pack(0)
        pltpu.async_copy(o_v.at[0], out_slice(last), so[0])
        wait_wb(last, 0)
        wait_wb(last - 1, 1)

    return gather_mul


_gather_mul_cache = []


def _gather_mul(z, idx0, idx1):
    if not _gather_mul_cache:
        _gather_mul_cache.append(_make_gather_mul())
    return _gather_mul_cache[0](z, idx0, idx1)


BE = 8000


def _mlp_body(x_ref, w1_ref, b1_ref, w2t_ref, b2_ref, o_ref):
    x = x_ref[...].astype(jnp.bfloat16)
    h = jnp.dot(x, w1_ref[...], preferred_element_type=jnp.float32)
    h = jnp.maximum(h + b1_ref[...], 0.0)
    logits = jnp.sum(h * w2t_ref[...], axis=1, keepdims=True) + b2_ref[...]
    o_ref[...] = jax.nn.sigmoid(logits)


def _mlp(x, W1, b1, W2, b2):
    grid = N_EDGES // BE
    return pl.pallas_call(
        _mlp_body,
        grid=(grid,),
        in_specs=[
            pl.BlockSpec((BE, D), lambda i: (i, 0)),
            pl.BlockSpec((D, HID), lambda i: (0, 0)),
            pl.BlockSpec((1, HID), lambda i: (0, 0)),
            pl.BlockSpec((1, HID), lambda i: (0, 0)),
            pl.BlockSpec((1, 1), lambda i: (0, 0)),
        ],
        out_specs=pl.BlockSpec((BE, 1), lambda i: (i, 0)),
        out_shape=jax.ShapeDtypeStruct((N_EDGES, 1), jnp.float32),
        compiler_params=pltpu.CompilerParams(
            dimension_semantics=("arbitrary",),
        ),
    )(x, W1, b1, W2, b2)


def kernel(z, edge, W1, b1, W2, b2):
    edge = edge.astype(jnp.int32)
    idx0 = edge[0].reshape(NW, N_CHUNKS, CHUNK)
    idx1 = edge[1].reshape(NW, N_CHUNKS, CHUNK)
    x = _gather_mul(z, idx0, idx1)
    return _mlp(x, W1.astype(jnp.bfloat16), b1.reshape(1, HID),
                W2.reshape(1, HID), b2.reshape(1, 1))


# speedup vs baseline: 2.5071x; 1.0051x over previous
import functools

import jax
import jax.numpy as jnp
import numpy as np
from jax import lax
from jax.experimental import pallas as pl
from jax.experimental.pallas import tpu as pltpu
from jax.experimental.pallas import tpu_sc as plsc

N_NODES = 10000
N_EDGES = 160000
D = 256
HID = 512

NC = 2
NS = 16
NW = NC * NS
E_PER_W = N_EDGES // NW
CHUNK = 40
N_CHUNKS = E_PER_W // CHUNK
LANES = 16

def _make_gather_mul():
    mesh = plsc.VectorSubcoreMesh(core_axis_name="c", subcore_axis_name="s")

    @functools.partial(
        pl.kernel,
        mesh=mesh,
        out_type=jax.ShapeDtypeStruct((N_EDGES, D), jnp.float32),
        scratch_types=[
            pltpu.VMEM((N_CHUNKS, CHUNK), jnp.int32),
            pltpu.VMEM((N_CHUNKS, CHUNK), jnp.int32),
            pltpu.VMEM((2, CHUNK, D), jnp.float32),
            pltpu.VMEM((2, CHUNK, D), jnp.float32),
            pltpu.VMEM((2, CHUNK, D), jnp.float32),
            pltpu.SemaphoreType.DMA,
            pltpu.SemaphoreType.DMA,
            pltpu.SemaphoreType.DMA,
            pltpu.SemaphoreType.DMA,
            pltpu.SemaphoreType.DMA,
            pltpu.SemaphoreType.DMA,
        ],
    )
    def gather_mul(z_hbm, idx0_hbm, idx1_hbm, out_hbm,
                   idx0_v, idx1_v, a_v, b_v, o_v,
                   sa0, sa1, sb0, sb1, so0, so1):
        wid = lax.axis_index("s") * NC + lax.axis_index("c")
        base = wid * E_PER_W
        sa = (sa0, sa1)
        sb = (sb0, sb1)
        so = (so0, so1)

        pltpu.sync_copy(idx0_hbm.at[wid], idx0_v)
        pltpu.sync_copy(idx1_hbm.at[wid], idx1_v)

        def start_gather(ci, p):
            pltpu.async_copy(z_hbm.at[idx0_v.at[ci]], a_v.at[p], sa[p])
            pltpu.async_copy(z_hbm.at[idx1_v.at[ci]], b_v.at[p], sb[p])

        def wait_gather(ci, p):
            pltpu.make_async_copy(z_hbm.at[idx0_v.at[ci]], a_v.at[p], sa[p]).wait()
            pltpu.make_async_copy(z_hbm.at[idx1_v.at[ci]], b_v.at[p], sb[p]).wait()

        def out_slice(ci):
            return out_hbm.at[pl.ds(base + ci * CHUNK, CHUNK)]

        def mul_pack(p):
            def row_body(r, c):
                for k in range(D // LANES):
                    sl = pl.ds(LANES * k, LANES)
                    o_v[p, r, sl] = a_v[p, r, sl] * b_v[p, r, sl]
                return c
            lax.fori_loop(0, CHUNK, row_body, 0)

        def wait_wb(ci, p):
            pltpu.make_async_copy(o_v.at[p], out_slice(ci), so[p]).wait()

        def process(ci, p, first):
            if not first:
                wait_wb(ci - 2, p)
            mul_pack(p)
            pltpu.async_copy(o_v.at[p], out_slice(ci), so[p])

        start_gather(0, 0)

        def pair_body(i, carry):
            g0 = 2 * i
            wait_gather(g0, 0)
            start_gather(g0 + 1, 1)

            @pl.when(i > 0)
            def _():
                wait_wb(g0 - 2, 0)
            mul_pack(0)
            pltpu.async_copy(o_v.at[0], out_slice(g0), so[0])

            wait_gather(g0 + 1, 1)
            start_gather(g0 + 2, 0)

            @pl.when(i > 0)
            def _():
                wait_wb(g0 - 1, 1)
            mul_pack(1)
            pltpu.async_copy(o_v.at[1], out_slice(g0 + 1), so[1])
            return carry

        lax.fori_loop(0, (N_CHUNKS - 1) // 2, pair_body, 0)

        last = N_CHUNKS - 1
        wait_gather(last, 0)
        wait_wb(last - 2, 0)
        mul_pack(0)
        pltpu.async_copy(o_v.at[0], out_slice(last), so[0])
        wait_wb(last, 0)
        wait_wb(last - 1, 1)

    return gather_mul


_gather_mul_cache = []


def _gather_mul(z, idx0, idx1):
    if not _gather_mul_cache:
        _gather_mul_cache.append(_make_gather_mul())
    return _gather_mul_cache[0](z, idx0, idx1)


BE = 16000


def _mlp_body(x_ref, w1_ref, b1_ref, w2t_ref, b2_ref, o_ref):
    x = x_ref[...].astype(jnp.bfloat16)
    h = jnp.dot(x, w1_ref[...], preferred_element_type=jnp.float32)
    h = jnp.maximum(h + b1_ref[...], 0.0)
    logits = jnp.sum(h * w2t_ref[...], axis=1, keepdims=True) + b2_ref[...]
    o_ref[...] = jax.nn.sigmoid(logits)


def _mlp(x, W1, b1, W2, b2):
    grid = N_EDGES // BE
    return pl.pallas_call(
        _mlp_body,
        grid=(grid,),
        in_specs=[
            pl.BlockSpec((BE, D), lambda i: (i, 0)),
            pl.BlockSpec((D, HID), lambda i: (0, 0)),
            pl.BlockSpec((1, HID), lambda i: (0, 0)),
            pl.BlockSpec((1, HID), lambda i: (0, 0)),
            pl.BlockSpec((1, 1), lambda i: (0, 0)),
        ],
        out_specs=pl.BlockSpec((BE, 1), lambda i: (i, 0)),
        out_shape=jax.ShapeDtypeStruct((N_EDGES, 1), jnp.float32),
        compiler_params=pltpu.CompilerParams(
            dimension_semantics=("arbitrary",),
        ),
    )(x, W1, b1, W2, b2)


def kernel(z, edge, W1, b1, W2, b2):
    edge = edge.astype(jnp.int32)
    idx0 = edge[0].reshape(NW, N_CHUNKS, CHUNK)
    idx1 = edge[1].reshape(NW, N_CHUNKS, CHUNK)
    x = _gather_mul(z, idx0, idx1)
    return _mlp(x, W1.astype(jnp.bfloat16), b1.reshape(1, HID),
                W2.reshape(1, HID), b2.reshape(1, 1))


# speedup vs baseline: 2.5781x; 1.0283x over previous
import functools

import jax
import jax.numpy as jnp
import numpy as np
from jax import lax
from jax.experimental import pallas as pl
from jax.experimental.pallas import tpu as pltpu
from jax.experimental.pallas import tpu_sc as plsc

N_NODES = 10000
N_EDGES = 160000
D = 256
HID = 512

NC = 2
NS = 16
NW = NC * NS
NSPLIT = 5
E_SPLIT = N_EDGES // NSPLIT
E_PER_W = E_SPLIT // NW
CHUNK = 40
N_CHUNKS = E_PER_W // CHUNK
LANES = 16

def _make_gather_mul():
    mesh = plsc.VectorSubcoreMesh(core_axis_name="c", subcore_axis_name="s")

    @functools.partial(
        pl.kernel,
        mesh=mesh,
        out_type=jax.ShapeDtypeStruct((E_SPLIT, D), jnp.float32),
        scratch_types=[
            pltpu.VMEM((N_CHUNKS, CHUNK), jnp.int32),
            pltpu.VMEM((N_CHUNKS, CHUNK), jnp.int32),
            pltpu.VMEM((2, CHUNK, D), jnp.float32),
            pltpu.VMEM((2, CHUNK, D), jnp.float32),
            pltpu.VMEM((2, CHUNK, D), jnp.float32),
            pltpu.SemaphoreType.DMA,
            pltpu.SemaphoreType.DMA,
            pltpu.SemaphoreType.DMA,
            pltpu.SemaphoreType.DMA,
            pltpu.SemaphoreType.DMA,
            pltpu.SemaphoreType.DMA,
        ],
    )
    def gather_mul(z_hbm, idx0_hbm, idx1_hbm, out_hbm,
                   idx0_v, idx1_v, a_v, b_v, o_v,
                   sa0, sa1, sb0, sb1, so0, so1):
        wid = lax.axis_index("s") * NC + lax.axis_index("c")
        base = wid * E_PER_W
        sa = (sa0, sa1)
        sb = (sb0, sb1)
        so = (so0, so1)

        pltpu.sync_copy(idx0_hbm.at[wid], idx0_v)
        pltpu.sync_copy(idx1_hbm.at[wid], idx1_v)

        def start_gather(ci, p):
            pltpu.async_copy(z_hbm.at[idx0_v.at[ci]], a_v.at[p], sa[p])
            pltpu.async_copy(z_hbm.at[idx1_v.at[ci]], b_v.at[p], sb[p])

        def wait_gather(ci, p):
            pltpu.make_async_copy(z_hbm.at[idx0_v.at[ci]], a_v.at[p], sa[p]).wait()
            pltpu.make_async_copy(z_hbm.at[idx1_v.at[ci]], b_v.at[p], sb[p]).wait()

        def out_slice(ci):
            return out_hbm.at[pl.ds(base + ci * CHUNK, CHUNK)]

        def mul_pack(p):
            def row_body(r, c):
                for k in range(D // LANES):
                    sl = pl.ds(LANES * k, LANES)
                    o_v[p, r, sl] = a_v[p, r, sl] * b_v[p, r, sl]
                return c
            lax.fori_loop(0, CHUNK, row_body, 0)

        def wait_wb(ci, p):
            pltpu.make_async_copy(o_v.at[p], out_slice(ci), so[p]).wait()

        def process(ci, p, first):
            if not first:
                wait_wb(ci - 2, p)
            mul_pack(p)
            pltpu.async_copy(o_v.at[p], out_slice(ci), so[p])

        start_gather(0, 0)

        def pair_body(i, carry):
            g0 = 2 * i
            wait_gather(g0, 0)
            start_gather(g0 + 1, 1)

            @pl.when(i > 0)
            def _():
                wait_wb(g0 - 2, 0)
            mul_pack(0)
            pltpu.async_copy(o_v.at[0], out_slice(g0), so[0])

            wait_gather(g0 + 1, 1)
            start_gather(g0 + 2, 0)

            @pl.when(i > 0)
            def _():
                wait_wb(g0 - 1, 1)
            mul_pack(1)
            pltpu.async_copy(o_v.at[1], out_slice(g0 + 1), so[1])
            return carry

        lax.fori_loop(0, (N_CHUNKS - 1) // 2, pair_body, 0)

        last = N_CHUNKS - 1
        wait_gather(last, 0)
        wait_wb(last - 2, 0)
        mul_pack(0)
        pltpu.async_copy(o_v.at[0], out_slice(last), so[0])
        wait_wb(last, 0)
        wait_wb(last - 1, 1)

    return gather_mul


_gather_mul_cache = []


def _gather_mul(z, idx0, idx1):
    if not _gather_mul_cache:
        _gather_mul_cache.append(_make_gather_mul())
    return _gather_mul_cache[0](z, idx0, idx1)


BE = 8000


def _mlp_body(x_ref, w1_ref, b1_ref, w2t_ref, b2_ref, o_ref):
    x = x_ref[...].astype(jnp.bfloat16)
    h = jnp.dot(x, w1_ref[...], preferred_element_type=jnp.float32)
    h = jnp.maximum(h + b1_ref[...], 0.0)
    logits = jnp.sum(h * w2t_ref[...], axis=1, keepdims=True) + b2_ref[...]
    o_ref[...] = jax.nn.sigmoid(logits)


def _mlp(x, W1, b1, W2, b2):
    grid = E_SPLIT // BE
    return pl.pallas_call(
        _mlp_body,
        grid=(grid,),
        in_specs=[
            pl.BlockSpec((BE, D), lambda i: (i, 0)),
            pl.BlockSpec((D, HID), lambda i: (0, 0)),
            pl.BlockSpec((1, HID), lambda i: (0, 0)),
            pl.BlockSpec((1, HID), lambda i: (0, 0)),
            pl.BlockSpec((1, 1), lambda i: (0, 0)),
        ],
        out_specs=pl.BlockSpec((BE, 1), lambda i: (i, 0)),
        out_shape=jax.ShapeDtypeStruct((E_SPLIT, 1), jnp.float32),
        compiler_params=pltpu.CompilerParams(
            dimension_semantics=("arbitrary",),
        ),
    )(x, W1, b1, W2, b2)


def kernel(z, edge, W1, b1, W2, b2):
    edge = edge.astype(jnp.int32)
    idx0 = edge[0].reshape(NSPLIT, NW, N_CHUNKS, CHUNK)
    idx1 = edge[1].reshape(NSPLIT, NW, N_CHUNKS, CHUNK)
    w1 = W1.astype(jnp.bfloat16)
    b1r = b1.reshape(1, HID)
    w2r = W2.reshape(1, HID)
    b2r = b2.reshape(1, 1)
    outs = []
    for k in range(NSPLIT):
        x = _gather_mul(z, idx0[k], idx1[k])
        outs.append(_mlp(x, w1, b1r, w2r, b2r))
    return jnp.concatenate(outs, axis=0)
